# baseline (speedup 1.0000x reference)
import functools

import jax
import jax.numpy as jnp
from jax import lax
from jax.experimental import pallas as pl
from jax.experimental.pallas import tpu as pltpu
from jax.experimental.pallas import tpu_sc as plsc

N = 10000
E = 320000
D = 128

NC = 2
NS = 16
NW = NC * NS
CHUNK = 80
EPAD = 327680
CPW = EPAD // (NW * CHUNK)
NPAD = 10240
RPS = NPAD // NS

BR = 640
GRID = NPAD // BR

_f32 = jnp.float32


@functools.lru_cache(maxsize=None)
def _make_sc_scatter(F):
  mesh = plsc.VectorSubcoreMesh(core_axis_name="c", subcore_axis_name="s", num_cores=NC, num_subcores=NS)

  @functools.partial(
      pl.kernel,
      out_type=jax.ShapeDtypeStruct((NC, NPAD, F), _f32),
      mesh=mesh,
      scratch_types=[
          pltpu.VMEM((CPW + 1, CHUNK), jnp.int32),
          pltpu.VMEM((CPW + 1, CHUNK), jnp.int32),
          pltpu.VMEM((CHUNK, F), _f32),
          pltpu.VMEM((CHUNK, F), _f32),
          pltpu.VMEM_SHARED((NPAD, F), _f32),
          pltpu.SemaphoreType.DMA,
          pltpu.SemaphoreType.DMA,
      ],
      compiler_params=pltpu.CompilerParams(use_tc_tiling_on_sc=False),
  )
  def k(h_hbm, srcp_hbm, dstp_hbm, zer_hbm, out_hbm, src_v, dst_v, rows0,
        rows1, acc, sem0, sem1):
    c = lax.axis_index("c")
    s = lax.axis_index("s")
    wid = s * NC + c
    pltpu.sync_copy(srcp_hbm.at[wid], src_v)
    pltpu.sync_copy(dstp_hbm.at[wid], dst_v)
    pltpu.sync_copy(zer_hbm, acc.at[pl.ds(s * RPS, RPS)])
    plsc.subcore_barrier()

    pltpu.async_copy(h_hbm.at[src_v.at[0]], rows0, sem0)

    def body(jj, carry):
      j0 = jj * 2
      pltpu.async_copy(h_hbm.at[src_v.at[j0 + 1]], rows1, sem1)
      pltpu.make_async_copy(h_hbm.at[src_v.at[j0]], rows0, sem0).wait()
      pltpu.sync_copy(rows0, acc.at[dst_v.at[j0]], add=True)
      pltpu.async_copy(h_hbm.at[src_v.at[j0 + 2]], rows0, sem0)
      pltpu.make_async_copy(h_hbm.at[src_v.at[j0 + 1]], rows1, sem1).wait()
      pltpu.sync_copy(rows1, acc.at[dst_v.at[j0 + 1]], add=True)
      return carry

    lax.fori_loop(0, CPW // 2, body, 0)
    pltpu.make_async_copy(h_hbm.at[src_v.at[CPW]], rows0, sem0).wait()
    plsc.subcore_barrier()
    pltpu.sync_copy(acc.at[pl.ds(s * RPS, RPS)],
                    out_hbm.at[c, pl.ds(s * RPS, RPS)])

  return k


_DEGW = 16


@functools.lru_cache(maxsize=None)
def _make_sc_deg():
  mesh = plsc.VectorSubcoreMesh(core_axis_name="c", subcore_axis_name="s", num_cores=NC, num_subcores=NS)

  @functools.partial(
      pl.kernel,
      out_type=jax.ShapeDtypeStruct((NC, NPAD, _DEGW), _f32),
      mesh=mesh,
      scratch_types=[
          pltpu.VMEM((CPW, CHUNK), jnp.int32),
          pltpu.VMEM((CHUNK, _DEGW), _f32),
          pltpu.VMEM_SHARED((NPAD, _DEGW), _f32),
      ],
      compiler_params=pltpu.CompilerParams(use_tc_tiling_on_sc=False),
  )
  def k(dstp_hbm, ones_hbm, zer_hbm, out_hbm, dst_v, ones_v, acc):
    c = lax.axis_index("c")
    s = lax.axis_index("s")
    wid = s * NC + c
    pltpu.sync_copy(dstp_hbm.at[wid, pl.ds(0, CPW)], dst_v)
    pltpu.sync_copy(ones_hbm, ones_v)
    pltpu.sync_copy(zer_hbm, acc.at[pl.ds(s * RPS, RPS)])
    plsc.subcore_barrier()

    def body(j, carry):
      pltpu.sync_copy(ones_v, acc.at[dst_v.at[j]], add=True)
      return carry

    lax.fori_loop(0, CPW, body, 0)
    plsc.subcore_barrier()
    pltpu.sync_copy(acc.at[pl.ds(s * RPS, RPS)],
                    out_hbm.at[c, pl.ds(s * RPS, RPS)])

  return k


def _full(shape):
  return pl.BlockSpec(shape, lambda i: tuple(0 for _ in shape))


def _rows(shape):
  if len(shape) == 3:
    return pl.BlockSpec(shape, lambda i: (0, i, 0))
  return pl.BlockSpec(shape, lambda i: (i, 0))


def _tc_pre(x, t, deg_p, w1a, w1b):
  def body(x_ref, t_ref, dp_ref, wa_ref, wb_ref, dis_ref, hs_ref):
    dp = dp_ref[...]
    deg = dp[0][:, 0:1] + dp[1][:, 0:1] + 1.0
    dis = lax.rsqrt(deg)
    h = jnp.dot(x_ref[...], wa_ref[...], preferred_element_type=_f32)
    h = h + t_ref[...] * wb_ref[...]
    dis_ref[...] = dis
    hs_ref[...] = dis * h

  return pl.pallas_call(
      body,
      grid=(GRID,),
      in_specs=[
          _rows((BR, D)),
          _rows((BR, 1)),
          _rows((NC, BR, _DEGW)),
          _full((D, 64)),
          _full((1, 64)),
      ],
      out_specs=[_rows((BR, 1)), _rows((BR, 64))],
      out_shape=[
          jax.ShapeDtypeStruct((NPAD, 1), _f32),
          jax.ShapeDtypeStruct((NPAD, 64), _f32),
      ],
  )(x, t, deg_p, w1a, w1b)


def _tc_dense(parts, dis, b, w, fouts):
  fins = [hs.shape[1] for _, hs in parts]
  fin = sum(fins)
  fout = sum(fouts)
  n_parts = len(parts)

  def body(*refs):
    in_refs = refs[:2 * n_parts]
    dis_ref, b_ref, w_ref = refs[2 * n_parts:2 * n_parts + 3]
    out_refs = refs[2 * n_parts + 3:]
    dis = dis_ref[...]
    segs = []
    for i in range(n_parts):
      p = in_refs[2 * i][...]
      hs = in_refs[2 * i + 1][...]
      segs.append(p[0] + p[1] + hs)
    agg = segs[0] if n_parts == 1 else jnp.concatenate(segs, axis=1)
    a = jax.nn.gelu(dis * agg + b_ref[...])
    h = jnp.dot(a, w_ref[...], preferred_element_type=_f32)
    hs_out = dis * h
    off = 0
    for r, f in zip(out_refs, fouts):
      r[...] = hs_out[:, off:off + f]
      off += f

  in_specs = []
  args = []
  for p, hs in parts:
    f = hs.shape[1]
    in_specs += [_rows((NC, BR, f)), _rows((BR, f))]
    args += [p, hs]
  in_specs += [_rows((BR, 1)), _full((1, fin)), _full((fin, fout))]
  args += [dis, b, w]

  return pl.pallas_call(
      body,
      grid=(GRID,),
      in_specs=in_specs,
      out_specs=[_rows((BR, f)) for f in fouts],
      out_shape=[jax.ShapeDtypeStruct((NPAD, f), _f32) for f in fouts],
  )(*args)


def _tc_final(p4, hs4, dis, b4, fw1, fb1, fw2, fb2, fw3, fb3):
  def body(p_ref, hs_ref, dis_ref, b_ref, w1_ref, c1_ref, w2_ref, c2_ref,
           w3_ref, c3_ref, out_ref):
    dis = dis_ref[...]
    p = p_ref[...]
    a = jax.nn.gelu(dis * (p[0] + p[1] + hs_ref[...]) + b_ref[...])
    z = jax.nn.gelu(
        jnp.dot(a, w1_ref[...], preferred_element_type=_f32) + c1_ref[...])
    z = jax.nn.gelu(
        jnp.dot(z, w2_ref[...], preferred_element_type=_f32) + c2_ref[...])
    out_ref[...] = (
        jnp.dot(z, w3_ref[...], preferred_element_type=_f32) + c3_ref[...])

  return pl.pallas_call(
      body,
      grid=(GRID,),
      in_specs=[
          _rows((NC, BR, 128)),
          _rows((BR, 128)),
          _rows((BR, 1)),
          _full((1, 128)),
          _full((128, 256)),
          _full((1, 256)),
          _full((256, 128)),
          _full((1, 128)),
          _full((128, 128)),
          _full((1, 128)),
      ],
      out_specs=_rows((BR, 128)),
      out_shape=jax.ShapeDtypeStruct((NPAD, 128), _f32),
  )(p4, hs4, dis, b4, fw1, fb1, fw2, fb2, fw3, fb3)


def kernel(x, adj, t, W1, b1, W2, b2, W3, b3, W4, b4,
           fw1, fb1, fw2, fb2, fw3, fb3):
  pad_i = jnp.full((EPAD - E,), N, dtype=jnp.int32)
  dummy = jnp.full((NW, 1, CHUNK), N, dtype=jnp.int32)
  srcp = jnp.concatenate(
      [jnp.concatenate([adj[0], pad_i]).reshape(NW, CPW, CHUNK), dummy], axis=1)
  dstp = jnp.concatenate(
      [jnp.concatenate([adj[1], pad_i]).reshape(NW, CPW, CHUNK), dummy], axis=1)

  ones_w = jnp.ones((CHUNK, _DEGW), _f32)
  zer_w = jnp.zeros((RPS, _DEGW), _f32)
  zer64 = jnp.zeros((RPS, 64), _f32)
  zer128 = jnp.zeros((RPS, 128), _f32)

  xp = jnp.zeros((NPAD, D), _f32).at[:N].set(x.astype(_f32))
  tp = jnp.zeros((NPAD, 1), _f32).at[:N, 0].set(t.astype(_f32))

  deg_p = _make_sc_deg()(dstp, ones_w, zer_w)
  dis, hs1 = _tc_pre(xp, tp, deg_p, W1[:D], W1[D:].reshape(1, 64))

  p1 = _make_sc_scatter(64)(hs1, srcp, dstp, zer64)
  hs2, = _tc_dense([(p1, hs1)], dis, b1.reshape(1, 64), W2, [128])

  p2 = _make_sc_scatter(128)(hs2, srcp, dstp, zer128)
  hs3a, hs3b = _tc_dense([(p2, hs2)], dis, b2.reshape(1, 128), W3,
                         [128, 128])

  p3a = _make_sc_scatter(128)(hs3a, srcp, dstp, zer128)
  p3b = _make_sc_scatter(128)(hs3b, srcp, dstp, zer128)
  hs4, = _tc_dense([(p3a, hs3a), (p3b, hs3b)], dis, b3.reshape(1, 256), W4,
                   [128])

  p4 = _make_sc_scatter(128)(hs4, srcp, dstp, zer128)
  out = _tc_final(p4, hs4, dis, b4.reshape(1, 128),
                  fw1, fb1.reshape(1, 256), fw2, fb2.reshape(1, 128),
                  fw3, fb3.reshape(1, 128))
  return out[:N]


# speedup vs baseline: 1.1743x; 1.1743x over previous
import functools

import jax
import jax.numpy as jnp
from jax import lax
from jax.experimental import pallas as pl
from jax.experimental.pallas import tpu as pltpu
from jax.experimental.pallas import tpu_sc as plsc

N = 10000
E = 320000
D = 128

NC = 2
NS = 16
NW = NC * NS
CHUNK = 128
EPAD = 327680
NCHUNKS = EPAD // CHUNK
CPW0 = 104
CPW1 = 56
NPAD = 10240
RPS = NPAD // NS

BR = 640
GRID = NPAD // BR

_f32 = jnp.float32


@functools.lru_cache(maxsize=None)
def _make_sc_scatter(F):
  mesh = plsc.VectorSubcoreMesh(core_axis_name="c", subcore_axis_name="s", num_cores=NC, num_subcores=NS)

  @functools.partial(
      pl.kernel,
      out_type=jax.ShapeDtypeStruct((NC, NPAD, F), _f32),
      mesh=mesh,
      scratch_types=[
          pltpu.VMEM((CPW0, CHUNK), jnp.int32),
          pltpu.VMEM((CPW0, CHUNK), jnp.int32),
          pltpu.VMEM((CHUNK, F), _f32),
          pltpu.VMEM_SHARED((NPAD, F), _f32),
          pltpu.SemaphoreType.DMA,
      ],
      compiler_params=pltpu.CompilerParams(use_tc_tiling_on_sc=False),
  )
  def k(h_hbm, srcp_hbm, dstp_hbm, zer_hbm, out_hbm, src_v, dst_v, rows_v,
        acc, sem):
    c = lax.axis_index("c")
    s = lax.axis_index("s")
    wid = s * NC + c
    pltpu.sync_copy(srcp_hbm.at[wid], src_v)
    pltpu.sync_copy(dstp_hbm.at[wid], dst_v)
    pltpu.sync_copy(zer_hbm, acc.at[pl.ds(s * RPS, RPS)])
    plsc.subcore_barrier()

    cnt = jnp.where(c == 0, CPW0, CPW1)

    def body(j, carry):
      pltpu.async_copy(h_hbm.at[src_v.at[j]], rows_v, sem).wait()
      pltpu.sync_copy(rows_v, acc.at[dst_v.at[j]], add=True)
      return carry

    lax.fori_loop(0, cnt, body, 0)
    plsc.subcore_barrier()
    pltpu.sync_copy(acc.at[pl.ds(s * RPS, RPS)],
                    out_hbm.at[c, pl.ds(s * RPS, RPS)])

  return k


_DEGW = 16


@functools.lru_cache(maxsize=None)
def _make_sc_deg():
  mesh = plsc.VectorSubcoreMesh(core_axis_name="c", subcore_axis_name="s", num_cores=NC, num_subcores=NS)

  @functools.partial(
      pl.kernel,
      out_type=jax.ShapeDtypeStruct((NC, NPAD, _DEGW), _f32),
      mesh=mesh,
      scratch_types=[
          pltpu.VMEM((CPW0, CHUNK), jnp.int32),
          pltpu.VMEM((CHUNK, _DEGW), _f32),
          pltpu.VMEM_SHARED((NPAD, _DEGW), _f32),
      ],
      compiler_params=pltpu.CompilerParams(use_tc_tiling_on_sc=False),
  )
  def k(dstp_hbm, ones_hbm, zer_hbm, out_hbm, dst_v, ones_v, acc):
    c = lax.axis_index("c")
    s = lax.axis_index("s")
    wid = s * NC + c
    pltpu.sync_copy(dstp_hbm.at[wid], dst_v)
    pltpu.sync_copy(ones_hbm, ones_v)
    pltpu.sync_copy(zer_hbm, acc.at[pl.ds(s * RPS, RPS)])
    plsc.subcore_barrier()

    def body(j, carry):
      pltpu.sync_copy(ones_v, acc.at[dst_v.at[j]], add=True)
      return carry

    lax.fori_loop(0, CPW0, body, 0)
    plsc.subcore_barrier()
    pltpu.sync_copy(acc.at[pl.ds(s * RPS, RPS)],
                    out_hbm.at[c, pl.ds(s * RPS, RPS)])

  return k


def _full(shape):
  return pl.BlockSpec(shape, lambda i: tuple(0 for _ in shape))


def _rows(shape):
  if len(shape) == 3:
    return pl.BlockSpec(shape, lambda i: (0, i, 0))
  return pl.BlockSpec(shape, lambda i: (i, 0))


def _tc_pre(x, t, deg_p, w1a, w1b):
  def body(x_ref, t_ref, dp_ref, wa_ref, wb_ref, dis_ref, hs_ref):
    dp = dp_ref[...]
    deg = dp[0][:, 0:1] + dp[1][:, 0:1] + 1.0
    dis = lax.rsqrt(deg)
    h = jnp.dot(x_ref[...], wa_ref[...], preferred_element_type=_f32)
    h = h + t_ref[...] * wb_ref[...]
    dis_ref[...] = dis
    hs_ref[...] = dis * h

  return pl.pallas_call(
      body,
      grid=(GRID,),
      in_specs=[
          _rows((BR, D)),
          _rows((BR, 1)),
          _rows((NC, BR, _DEGW)),
          _full((D, 64)),
          _full((1, 64)),
      ],
      out_specs=[_rows((BR, 1)), _rows((BR, 64))],
      out_shape=[
          jax.ShapeDtypeStruct((NPAD, 1), _f32),
          jax.ShapeDtypeStruct((NPAD, 64), _f32),
      ],
  )(x, t, deg_p, w1a, w1b)


def _tc_dense(parts, dis, b, w, fouts):
  fins = [hs.shape[1] for _, hs in parts]
  fin = sum(fins)
  fout = sum(fouts)
  n_parts = len(parts)

  def body(*refs):
    in_refs = refs[:2 * n_parts]
    dis_ref, b_ref, w_ref = refs[2 * n_parts:2 * n_parts + 3]
    out_refs = refs[2 * n_parts + 3:]
    dis = dis_ref[...]
    segs = []
    for i in range(n_parts):
      p = in_refs[2 * i][...]
      hs = in_refs[2 * i + 1][...]
      segs.append(p[0] + p[1] + hs)
    agg = segs[0] if n_parts == 1 else jnp.concatenate(segs, axis=1)
    a = jax.nn.gelu(dis * agg + b_ref[...])
    h = jnp.dot(a, w_ref[...], preferred_element_type=_f32)
    hs_out = dis * h
    off = 0
    for r, f in zip(out_refs, fouts):
      r[...] = hs_out[:, off:off + f]
      off += f

  in_specs = []
  args = []
  for p, hs in parts:
    f = hs.shape[1]
    in_specs += [_rows((NC, BR, f)), _rows((BR, f))]
    args += [p, hs]
  in_specs += [_rows((BR, 1)), _full((1, fin)), _full((fin, fout))]
  args += [dis, b, w]

  return pl.pallas_call(
      body,
      grid=(GRID,),
      in_specs=in_specs,
      out_specs=[_rows((BR, f)) for f in fouts],
      out_shape=[jax.ShapeDtypeStruct((NPAD, f), _f32) for f in fouts],
  )(*args)


def _tc_final(p4, hs4, dis, b4, fw1, fb1, fw2, fb2, fw3, fb3):
  def body(p_ref, hs_ref, dis_ref, b_ref, w1_ref, c1_ref, w2_ref, c2_ref,
           w3_ref, c3_ref, out_ref):
    dis = dis_ref[...]
    p = p_ref[...]
    a = jax.nn.gelu(dis * (p[0] + p[1] + hs_ref[...]) + b_ref[...])
    z = jax.nn.gelu(
        jnp.dot(a, w1_ref[...], preferred_element_type=_f32) + c1_ref[...])
    z = jax.nn.gelu(
        jnp.dot(z, w2_ref[...], preferred_element_type=_f32) + c2_ref[...])
    out_ref[...] = (
        jnp.dot(z, w3_ref[...], preferred_element_type=_f32) + c3_ref[...])

  return pl.pallas_call(
      body,
      grid=(GRID,),
      in_specs=[
          _rows((NC, BR, 128)),
          _rows((BR, 128)),
          _rows((BR, 1)),
          _full((1, 128)),
          _full((128, 256)),
          _full((1, 256)),
          _full((256, 128)),
          _full((1, 128)),
          _full((128, 128)),
          _full((1, 128)),
      ],
      out_specs=_rows((BR, 128)),
      out_shape=jax.ShapeDtypeStruct((NPAD, 128), _f32),
  )(p4, hs4, dis, b4, fw1, fb1, fw2, fb2, fw3, fb3)


def kernel(x, adj, t, W1, b1, W2, b2, W3, b3, W4, b4,
           fw1, fb1, fw2, fb2, fw3, fb3):
  pad_i = jnp.full((EPAD - E,), N, dtype=jnp.int32)

  def _slab(idx):
    ch = jnp.concatenate([idx, pad_i]).reshape(NCHUNKS, CHUNK)
    fill = jnp.full((CPW0 - CPW1, CHUNK), N, dtype=jnp.int32)
    slabs = []
    off = 0
    for w in range(NW):
      cnt = CPW0 if w % NC == 0 else CPW1
      blk = ch[off:off + cnt]
      off += cnt
      if cnt < CPW0:
        blk = jnp.concatenate([blk, fill])
      slabs.append(blk)
    return jnp.stack(slabs)

  srcp = _slab(adj[0])
  dstp = _slab(adj[1])

  ones_w = jnp.ones((CHUNK, _DEGW), _f32)
  zer_w = jnp.zeros((RPS, _DEGW), _f32)
  zer64 = jnp.zeros((RPS, 64), _f32)
  zer128 = jnp.zeros((RPS, 128), _f32)

  xp = jnp.zeros((NPAD, D), _f32).at[:N].set(x.astype(_f32))
  tp = jnp.zeros((NPAD, 1), _f32).at[:N, 0].set(t.astype(_f32))

  deg_p = _make_sc_deg()(dstp, ones_w, zer_w)
  dis, hs1 = _tc_pre(xp, tp, deg_p, W1[:D], W1[D:].reshape(1, 64))

  p1 = _make_sc_scatter(64)(hs1, srcp, dstp, zer64)
  hs2, = _tc_dense([(p1, hs1)], dis, b1.reshape(1, 64), W2, [128])

  p2 = _make_sc_scatter(128)(hs2, srcp, dstp, zer128)
  hs3a, hs3b = _tc_dense([(p2, hs2)], dis, b2.reshape(1, 128), W3,
                         [128, 128])

  p3a = _make_sc_scatter(128)(hs3a, srcp, dstp, zer128)
  p3b = _make_sc_scatter(128)(hs3b, srcp, dstp, zer128)
  hs4, = _tc_dense([(p3a, hs3a), (p3b, hs3b)], dis, b3.reshape(1, 256), W4,
                   [128])

  p4 = _make_sc_scatter(128)(hs4, srcp, dstp, zer128)
  out = _tc_final(p4, hs4, dis, b4.reshape(1, 128),
                  fw1, fb1.reshape(1, 256), fw2, fb2.reshape(1, 128),
                  fw3, fb3.reshape(1, 128))
  return out[:N]


# speedup vs baseline: 1.4427x; 1.2286x over previous
import functools

import jax
import jax.numpy as jnp
from jax import lax
from jax.experimental import pallas as pl
from jax.experimental.pallas import tpu as pltpu
from jax.experimental.pallas import tpu_sc as plsc

N = 10000
E = 320000
D = 128

NC = 2
NS = 16
NW = NC * NS
CHUNK = 128
EPAD = 327680
CPW = EPAD // (NW * CHUNK)
NPAD = 10240
RPS = NPAD // NS

BR = 640
GRID = NPAD // BR

_f32 = jnp.float32
_bf16 = jnp.bfloat16


def _widen_rows(bf_ref, f_ref, F):
  mask = jnp.int32(-65536)

  def row(r, carry):
    for u in range(F // 32):
      v = plsc.bitcast(bf_ref[r, pl.ds(u * 32, 32)], jnp.int32)
      lo = plsc.bitcast(lax.shift_left(v, 16), _f32)
      hi = plsc.bitcast(jnp.bitwise_and(v, mask), _f32)
      f_ref[r, pl.ds(u * 32, 16)] = lo
      f_ref[r, pl.ds(u * 32 + 16, 16)] = hi
    return carry

  lax.fori_loop(0, CHUNK, row, 0)


@functools.lru_cache(maxsize=None)
def _make_sc_scatter(F):
  mesh = plsc.VectorSubcoreMesh(core_axis_name="c", subcore_axis_name="s",
                                num_cores=NC, num_subcores=NS)

  @functools.partial(
      pl.kernel,
      out_type=jax.ShapeDtypeStruct((NC, NPAD, F), _f32),
      mesh=mesh,
      scratch_types=[
          pltpu.VMEM((CPW // 2 + 1, CHUNK), jnp.int32),
          pltpu.VMEM((CPW // 2, CHUNK), jnp.int32),
          pltpu.VMEM((CHUNK, F), _bf16),
          pltpu.VMEM((CHUNK, F), _bf16),
          pltpu.VMEM((CHUNK, F), _f32),
          pltpu.VMEM_SHARED((NPAD, F), _f32),
          pltpu.SemaphoreType.DMA,
          pltpu.SemaphoreType.DMA,
      ],
      compiler_params=pltpu.CompilerParams(use_tc_tiling_on_sc=False,
                                           needs_layout_passes=False),
  )
  def k(h_hbm, srcp_hbm, dstp_hbm, zer_hbm, out_hbm, src_v, dst_v, bf0, bf1,
        rows_f, acc, sem0, sem1):
    c = lax.axis_index("c")
    s = lax.axis_index("s")
    wid = s * NC + c
    half = CPW // 2
    pltpu.sync_copy(zer_hbm, acc.at[pl.ds(s * RPS, RPS)])
    plsc.subcore_barrier()

    for ph in range(2):
      pltpu.sync_copy(srcp_hbm.at[wid, pl.ds(ph * half, half + 1)], src_v)
      pltpu.sync_copy(dstp_hbm.at[wid, pl.ds(ph * half, half)], dst_v)

      pltpu.async_copy(h_hbm.at[src_v.at[0]], bf0, sem0)

      def body(jj, carry):
        j0 = jj * 2
        pltpu.async_copy(h_hbm.at[src_v.at[j0 + 1]], bf1, sem1)
        pltpu.make_async_copy(h_hbm.at[src_v.at[j0]], bf0, sem0).wait()
        _widen_rows(bf0, rows_f, F)
        pltpu.sync_copy(rows_f, acc.at[dst_v.at[j0]], add=True)
        pltpu.async_copy(h_hbm.at[src_v.at[j0 + 2]], bf0, sem0)
        pltpu.make_async_copy(h_hbm.at[src_v.at[j0 + 1]], bf1, sem1).wait()
        _widen_rows(bf1, rows_f, F)
        pltpu.sync_copy(rows_f, acc.at[dst_v.at[j0 + 1]], add=True)
        return carry

      lax.fori_loop(0, half // 2, body, 0)
      pltpu.make_async_copy(h_hbm.at[src_v.at[half]], bf0, sem0).wait()

    plsc.subcore_barrier()
    pltpu.sync_copy(acc.at[pl.ds(s * RPS, RPS)],
                    out_hbm.at[c, pl.ds(s * RPS, RPS)])

  return k


_DEGW = 16


@functools.lru_cache(maxsize=None)
def _make_sc_deg():
  mesh = plsc.VectorSubcoreMesh(core_axis_name="c", subcore_axis_name="s",
                                num_cores=NC, num_subcores=NS)

  @functools.partial(
      pl.kernel,
      out_type=jax.ShapeDtypeStruct((NC, NPAD, _DEGW), _f32),
      mesh=mesh,
      scratch_types=[
          pltpu.VMEM((CPW + 1, CHUNK), jnp.int32),
          pltpu.VMEM((CHUNK, _DEGW), _f32),
          pltpu.VMEM_SHARED((NPAD, _DEGW), _f32),
      ],
      compiler_params=pltpu.CompilerParams(use_tc_tiling_on_sc=False),
  )
  def k(dstp_hbm, ones_hbm, zer_hbm, out_hbm, dst_v, ones_v, acc):
    c = lax.axis_index("c")
    s = lax.axis_index("s")
    wid = s * NC + c
    pltpu.sync_copy(dstp_hbm.at[wid], dst_v)
    pltpu.sync_copy(ones_hbm, ones_v)
    pltpu.sync_copy(zer_hbm, acc.at[pl.ds(s * RPS, RPS)])
    plsc.subcore_barrier()

    def body(j, carry):
      pltpu.sync_copy(ones_v, acc.at[dst_v.at[j]], add=True)
      return carry

    lax.fori_loop(0, CPW, body, 0)
    plsc.subcore_barrier()
    pltpu.sync_copy(acc.at[pl.ds(s * RPS, RPS)],
                    out_hbm.at[c, pl.ds(s * RPS, RPS)])

  return k


def _full(shape):
  return pl.BlockSpec(shape, lambda i: tuple(0 for _ in shape))


def _rows(shape):
  if len(shape) == 3:
    return pl.BlockSpec(shape, lambda i: (0, i, 0))
  return pl.BlockSpec(shape, lambda i: (i, 0))


def _interleave(hs_bf):
  n, f = hs_bf.shape
  return hs_bf.reshape(n, f // 32, 2, 16).swapaxes(-1, -2).reshape(n, f)


def _tc_pre(x, t, deg_p, w1a, w1b):
  def body(x_ref, t_ref, dp_ref, wa_ref, wb_ref, dis_ref, hs_ref, hsb_ref):
    dp = dp_ref[...]
    deg = dp[0][:, 0:1] + dp[1][:, 0:1] + 1.0
    dis = lax.rsqrt(deg)
    h = jnp.dot(x_ref[...], wa_ref[...], preferred_element_type=_f32)
    h = h + t_ref[...] * wb_ref[...]
    hs = dis * h
    dis_ref[...] = dis
    hs_ref[...] = hs
    hsb_ref[...] = hs.astype(_bf16)

  return pl.pallas_call(
      body,
      grid=(GRID,),
      in_specs=[
          _rows((BR, D)),
          _rows((BR, 1)),
          _rows((NC, BR, _DEGW)),
          _full((D, 64)),
          _full((1, 64)),
      ],
      out_specs=[_rows((BR, 1)), _rows((BR, 64)), _rows((BR, 64))],
      out_shape=[
          jax.ShapeDtypeStruct((NPAD, 1), _f32),
          jax.ShapeDtypeStruct((NPAD, 64), _f32),
          jax.ShapeDtypeStruct((NPAD, 64), _bf16),
      ],
  )(x, t, deg_p, w1a, w1b)


def _tc_dense(parts, dis, b, w, fouts):
  fins = [hs.shape[1] for _, hs in parts]
  fin = sum(fins)
  fout = sum(fouts)
  n_parts = len(parts)
  n_out = len(fouts)

  def body(*refs):
    in_refs = refs[:2 * n_parts]
    dis_ref, b_ref, w_ref = refs[2 * n_parts:2 * n_parts + 3]
    out_refs = refs[2 * n_parts + 3:]
    dis = dis_ref[...]
    segs = []
    for i in range(n_parts):
      p = in_refs[2 * i][...]
      hs = in_refs[2 * i + 1][...]
      segs.append(p[0] + p[1] + hs)
    agg = segs[0] if n_parts == 1 else jnp.concatenate(segs, axis=1)
    a = jax.nn.gelu(dis * agg + b_ref[...])
    h = jnp.dot(a, w_ref[...], preferred_element_type=_f32)
    hs_out = dis * h
    off = 0
    for i, f in enumerate(fouts):
      blk = hs_out[:, off:off + f]
      out_refs[i][...] = blk
      out_refs[n_out + i][...] = blk.astype(_bf16)
      off += f

  in_specs = []
  args = []
  for p, hs in parts:
    f = hs.shape[1]
    in_specs += [_rows((NC, BR, f)), _rows((BR, f))]
    args += [p, hs]
  in_specs += [_rows((BR, 1)), _full((1, fin)), _full((fin, fout))]
  args += [dis, b, w]

  return pl.pallas_call(
      body,
      grid=(GRID,),
      in_specs=in_specs,
      out_specs=([_rows((BR, f)) for f in fouts] +
                 [_rows((BR, f)) for f in fouts]),
      out_shape=([jax.ShapeDtypeStruct((NPAD, f), _f32) for f in fouts] +
                 [jax.ShapeDtypeStruct((NPAD, f), _bf16) for f in fouts]),
  )(*args)


def _tc_final(p4, hs4, dis, b4, fw1, fb1, fw2, fb2, fw3, fb3):
  def body(p_ref, hs_ref, dis_ref, b_ref, w1_ref, c1_ref, w2_ref, c2_ref,
           w3_ref, c3_ref, out_ref):
    dis = dis_ref[...]
    p = p_ref[...]
    a = jax.nn.gelu(dis * (p[0] + p[1] + hs_ref[...]) + b_ref[...])
    z = jax.nn.gelu(
        jnp.dot(a, w1_ref[...], preferred_element_type=_f32) + c1_ref[...])
    z = jax.nn.gelu(
        jnp.dot(z, w2_ref[...], preferred_element_type=_f32) + c2_ref[...])
    out_ref[...] = (
        jnp.dot(z, w3_ref[...], preferred_element_type=_f32) + c3_ref[...])

  return pl.pallas_call(
      body,
      grid=(GRID,),
      in_specs=[
          _rows((NC, BR, 128)),
          _rows((BR, 128)),
          _rows((BR, 1)),
          _full((1, 128)),
          _full((128, 256)),
          _full((1, 256)),
          _full((256, 128)),
          _full((1, 128)),
          _full((128, 128)),
          _full((1, 128)),
      ],
      out_specs=_rows((BR, 128)),
      out_shape=jax.ShapeDtypeStruct((NPAD, 128), _f32),
  )(p4, hs4, dis, b4, fw1, fb1, fw2, fb2, fw3, fb3)


def kernel(x, adj, t, W1, b1, W2, b2, W3, b3, W4, b4,
           fw1, fb1, fw2, fb2, fw3, fb3):
  pad_i = jnp.full((EPAD - E,), N, dtype=jnp.int32)
  dummy = jnp.full((NW, 1, CHUNK), N, dtype=jnp.int32)

  def _slab(idx):
    base = jnp.concatenate([idx, pad_i]).reshape(NW, CPW, CHUNK)
    return jnp.concatenate([base, dummy], axis=1)

  srcp = _slab(adj[0])
  dstp = _slab(adj[1])

  ones_w = jnp.ones((CHUNK, _DEGW), _f32)
  zer_w = jnp.zeros((RPS, _DEGW), _f32)
  zer64 = jnp.zeros((RPS, 64), _f32)
  zer128 = jnp.zeros((RPS, 128), _f32)

  xp = jnp.zeros((NPAD, D), _f32).at[:N].set(x.astype(_f32))
  tp = jnp.zeros((NPAD, 1), _f32).at[:N, 0].set(t.astype(_f32))

  deg_p = _make_sc_deg()(dstp, ones_w, zer_w)
  dis, hs1, hs1b = _tc_pre(xp, tp, deg_p, W1[:D], W1[D:].reshape(1, 64))

  p1 = _make_sc_scatter(64)(_interleave(hs1b), srcp, dstp, zer64)
  hs2, hs2b = _tc_dense([(p1, hs1)], dis, b1.reshape(1, 64), W2, [128])

  p2 = _make_sc_scatter(128)(_interleave(hs2b), srcp, dstp, zer128)
  hs3a, hs3b, hs3ab, hs3bb = _tc_dense([(p2, hs2)], dis, b2.reshape(1, 128),
                                       W3, [128, 128])

  p3a = _make_sc_scatter(128)(_interleave(hs3ab), srcp, dstp, zer128)
  p3b = _make_sc_scatter(128)(_interleave(hs3bb), srcp, dstp, zer128)
  hs4, hs4b = _tc_dense([(p3a, hs3a), (p3b, hs3b)], dis, b3.reshape(1, 256),
                        W4, [128])

  p4 = _make_sc_scatter(128)(_interleave(hs4b), srcp, dstp, zer128)
  out = _tc_final(p4, hs4, dis, b4.reshape(1, 128),
                  fw1, fb1.reshape(1, 256), fw2, fb2.reshape(1, 128),
                  fw3, fb3.reshape(1, 128))
  return out[:N]


# speedup vs baseline: 1.5850x; 1.0986x over previous
import functools

import jax
import jax.numpy as jnp
from jax import lax
from jax.experimental import pallas as pl
from jax.experimental.pallas import tpu as pltpu
from jax.experimental.pallas import tpu_sc as plsc

N = 10000
E = 320000
D = 128

NC = 2
NS = 16
NW = NC * NS
CHUNK = 128
EPAD = 327680
CPW = EPAD // (NW * CHUNK)
NPAD = 10240
RPS = NPAD // NS

BR = 640
GRID = NPAD // BR

_f32 = jnp.float32
_bf16 = jnp.bfloat16


def _widen_rows(bf_ref, f_ref, F):
  mask = jnp.int32(-65536)

  @plsc.parallel_loop(0, CHUNK, 1, unroll=4)
  def _row(r):
    for u in range(F // 32):
      v = plsc.bitcast(bf_ref[r, pl.ds(u * 32, 32)], jnp.int32)
      lo = plsc.bitcast(lax.shift_left(v, 16), _f32)
      hi = plsc.bitcast(jnp.bitwise_and(v, mask), _f32)
      f_ref[r, pl.ds(u * 32, 16)] = lo
      f_ref[r, pl.ds(u * 32 + 16, 16)] = hi


@functools.lru_cache(maxsize=None)
def _make_sc_scatter(F):
  mesh = plsc.VectorSubcoreMesh(core_axis_name="c", subcore_axis_name="s",
                                num_cores=NC, num_subcores=NS)

  @functools.partial(
      pl.kernel,
      out_type=jax.ShapeDtypeStruct((NC, NPAD, F), _f32),
      mesh=mesh,
      scratch_types=[
          pltpu.VMEM((CPW // 2 + 1, CHUNK), jnp.int32),
          pltpu.VMEM((CPW // 2, CHUNK), jnp.int32),
          pltpu.VMEM((CHUNK, F), _bf16),
          pltpu.VMEM((CHUNK, F), _bf16),
          pltpu.VMEM((CHUNK, F), _f32),
          pltpu.VMEM_SHARED((NPAD, F), _f32),
          pltpu.SemaphoreType.DMA,
          pltpu.SemaphoreType.DMA,
      ],
      compiler_params=pltpu.CompilerParams(use_tc_tiling_on_sc=False,
                                           needs_layout_passes=False),
  )
  def k(h_hbm, srcp_hbm, dstp_hbm, zer_hbm, out_hbm, src_v, dst_v, bf0, bf1,
        rows_f, acc, sem0, sem1):
    c = lax.axis_index("c")
    s = lax.axis_index("s")
    wid = s * NC + c
    half = CPW // 2
    pltpu.sync_copy(zer_hbm, acc.at[pl.ds(s * RPS, RPS)])
    plsc.subcore_barrier()

    for ph in range(2):
      pltpu.sync_copy(srcp_hbm.at[wid, pl.ds(ph * half, half + 1)], src_v)
      pltpu.sync_copy(dstp_hbm.at[wid, pl.ds(ph * half, half)], dst_v)

      pltpu.async_copy(h_hbm.at[src_v.at[0]], bf0, sem0)

      def body(jj, carry):
        j0 = jj * 2
        pltpu.async_copy(h_hbm.at[src_v.at[j0 + 1]], bf1, sem1)
        pltpu.make_async_copy(h_hbm.at[src_v.at[j0]], bf0, sem0).wait()
        _widen_rows(bf0, rows_f, F)
        pltpu.sync_copy(rows_f, acc.at[dst_v.at[j0]], add=True)
        pltpu.async_copy(h_hbm.at[src_v.at[j0 + 2]], bf0, sem0)
        pltpu.make_async_copy(h_hbm.at[src_v.at[j0 + 1]], bf1, sem1).wait()
        _widen_rows(bf1, rows_f, F)
        pltpu.sync_copy(rows_f, acc.at[dst_v.at[j0 + 1]], add=True)
        return carry

      lax.fori_loop(0, half // 2, body, 0)
      pltpu.make_async_copy(h_hbm.at[src_v.at[half]], bf0, sem0).wait()

    plsc.subcore_barrier()
    pltpu.sync_copy(acc.at[pl.ds(s * RPS, RPS)],
                    out_hbm.at[c, pl.ds(s * RPS, RPS)])

  return k


_DEGW = 16


@functools.lru_cache(maxsize=None)
def _make_sc_deg():
  mesh = plsc.VectorSubcoreMesh(core_axis_name="c", subcore_axis_name="s",
                                num_cores=NC, num_subcores=NS)

  @functools.partial(
      pl.kernel,
      out_type=jax.ShapeDtypeStruct((NC, NPAD, _DEGW), _f32),
      mesh=mesh,
      scratch_types=[
          pltpu.VMEM((CPW + 1, CHUNK), jnp.int32),
          pltpu.VMEM((CHUNK, _DEGW), _f32),
          pltpu.VMEM_SHARED((NPAD, _DEGW), _f32),
      ],
      compiler_params=pltpu.CompilerParams(use_tc_tiling_on_sc=False),
  )
  def k(dstp_hbm, ones_hbm, zer_hbm, out_hbm, dst_v, ones_v, acc):
    c = lax.axis_index("c")
    s = lax.axis_index("s")
    wid = s * NC + c
    pltpu.sync_copy(dstp_hbm.at[wid], dst_v)
    pltpu.sync_copy(ones_hbm, ones_v)
    pltpu.sync_copy(zer_hbm, acc.at[pl.ds(s * RPS, RPS)])
    plsc.subcore_barrier()

    def body(j, carry):
      pltpu.sync_copy(ones_v, acc.at[dst_v.at[j]], add=True)
      return carry

    lax.fori_loop(0, CPW, body, 0)
    plsc.subcore_barrier()
    pltpu.sync_copy(acc.at[pl.ds(s * RPS, RPS)],
                    out_hbm.at[c, pl.ds(s * RPS, RPS)])

  return k


def _full(shape):
  return pl.BlockSpec(shape, lambda i: tuple(0 for _ in shape))


def _rows(shape):
  if len(shape) == 3:
    return pl.BlockSpec(shape, lambda i: (0, i, 0))
  return pl.BlockSpec(shape, lambda i: (i, 0))


def _interleave(hs_bf):
  n, f = hs_bf.shape
  return hs_bf.reshape(n, f // 32, 2, 16).swapaxes(-1, -2).reshape(n, f)


def _tc_pre(x, t, deg_p, w1a, w1b):
  def body(x_ref, t_ref, dp_ref, wa_ref, wb_ref, dis_ref, hs_ref, hsb_ref):
    dp = dp_ref[...]
    deg = dp[0][:, 0:1] + dp[1][:, 0:1] + 1.0
    dis = lax.rsqrt(deg)
    h = jnp.dot(x_ref[...], wa_ref[...], preferred_element_type=_f32)
    h = h + t_ref[...] * wb_ref[...]
    hs = dis * h
    dis_ref[...] = dis
    hs_ref[...] = hs
    hsb_ref[...] = hs.astype(_bf16)

  return pl.pallas_call(
      body,
      grid=(GRID,),
      in_specs=[
          _rows((BR, D)),
          _rows((BR, 1)),
          _rows((NC, BR, _DEGW)),
          _full((D, 64)),
          _full((1, 64)),
      ],
      out_specs=[_rows((BR, 1)), _rows((BR, 64)), _rows((BR, 64))],
      out_shape=[
          jax.ShapeDtypeStruct((NPAD, 1), _f32),
          jax.ShapeDtypeStruct((NPAD, 64), _f32),
          jax.ShapeDtypeStruct((NPAD, 64), _bf16),
      ],
  )(x, t, deg_p, w1a, w1b)


def _tc_dense(parts, dis, b, w, fouts):
  fins = [hs.shape[1] for _, hs in parts]
  fin = sum(fins)
  fout = sum(fouts)
  n_parts = len(parts)
  n_out = len(fouts)

  def body(*refs):
    in_refs = refs[:2 * n_parts]
    dis_ref, b_ref, w_ref = refs[2 * n_parts:2 * n_parts + 3]
    out_refs = refs[2 * n_parts + 3:]
    dis = dis_ref[...]
    segs = []
    for i in range(n_parts):
      p = in_refs[2 * i][...]
      hs = in_refs[2 * i + 1][...]
      segs.append(p[0] + p[1] + hs)
    agg = segs[0] if n_parts == 1 else jnp.concatenate(segs, axis=1)
    a = jax.nn.gelu(dis * agg + b_ref[...])
    h = jnp.dot(a, w_ref[...], preferred_element_type=_f32)
    hs_out = dis * h
    off = 0
    for i, f in enumerate(fouts):
      blk = hs_out[:, off:off + f]
      out_refs[i][...] = blk
      out_refs[n_out + i][...] = blk.astype(_bf16)
      off += f

  in_specs = []
  args = []
  for p, hs in parts:
    f = hs.shape[1]
    in_specs += [_rows((NC, BR, f)), _rows((BR, f))]
    args += [p, hs]
  in_specs += [_rows((BR, 1)), _full((1, fin)), _full((fin, fout))]
  args += [dis, b, w]

  return pl.pallas_call(
      body,
      grid=(GRID,),
      in_specs=in_specs,
      out_specs=([_rows((BR, f)) for f in fouts] +
                 [_rows((BR, f)) for f in fouts]),
      out_shape=([jax.ShapeDtypeStruct((NPAD, f), _f32) for f in fouts] +
                 [jax.ShapeDtypeStruct((NPAD, f), _bf16) for f in fouts]),
  )(*args)


def _tc_final(p4, hs4, dis, b4, fw1, fb1, fw2, fb2, fw3, fb3):
  def body(p_ref, hs_ref, dis_ref, b_ref, w1_ref, c1_ref, w2_ref, c2_ref,
           w3_ref, c3_ref, out_ref):
    dis = dis_ref[...]
    p = p_ref[...]
    a = jax.nn.gelu(dis * (p[0] + p[1] + hs_ref[...]) + b_ref[...])
    z = jax.nn.gelu(
        jnp.dot(a, w1_ref[...], preferred_element_type=_f32) + c1_ref[...])
    z = jax.nn.gelu(
        jnp.dot(z, w2_ref[...], preferred_element_type=_f32) + c2_ref[...])
    out_ref[...] = (
        jnp.dot(z, w3_ref[...], preferred_element_type=_f32) + c3_ref[...])

  return pl.pallas_call(
      body,
      grid=(GRID,),
      in_specs=[
          _rows((NC, BR, 128)),
          _rows((BR, 128)),
          _rows((BR, 1)),
          _full((1, 128)),
          _full((128, 256)),
          _full((1, 256)),
          _full((256, 128)),
          _full((1, 128)),
          _full((128, 128)),
          _full((1, 128)),
      ],
      out_specs=_rows((BR, 128)),
      out_shape=jax.ShapeDtypeStruct((NPAD, 128), _f32),
  )(p4, hs4, dis, b4, fw1, fb1, fw2, fb2, fw3, fb3)


def kernel(x, adj, t, W1, b1, W2, b2, W3, b3, W4, b4,
           fw1, fb1, fw2, fb2, fw3, fb3):
  pad_i = jnp.full((EPAD - E,), N, dtype=jnp.int32)
  dummy = jnp.full((NW, 1, CHUNK), N, dtype=jnp.int32)

  def _slab(idx):
    base = jnp.concatenate([idx, pad_i]).reshape(NW, CPW, CHUNK)
    return jnp.concatenate([base, dummy], axis=1)

  srcp = _slab(adj[0])
  dstp = _slab(adj[1])

  ones_w = jnp.ones((CHUNK, _DEGW), _f32)
  zer_w = jnp.zeros((RPS, _DEGW), _f32)
  zer64 = jnp.zeros((RPS, 64), _f32)
  zer128 = jnp.zeros((RPS, 128), _f32)

  xp = jnp.zeros((NPAD, D), _f32).at[:N].set(x.astype(_f32))
  tp = jnp.zeros((NPAD, 1), _f32).at[:N, 0].set(t.astype(_f32))

  deg_p = _make_sc_deg()(dstp, ones_w, zer_w)
  dis, hs1, hs1b = _tc_pre(xp, tp, deg_p, W1[:D], W1[D:].reshape(1, 64))

  p1 = _make_sc_scatter(64)(_interleave(hs1b), srcp, dstp, zer64)
  hs2, hs2b = _tc_dense([(p1, hs1)], dis, b1.reshape(1, 64), W2, [128])

  p2 = _make_sc_scatter(128)(_interleave(hs2b), srcp, dstp, zer128)
  hs3a, hs3b, hs3ab, hs3bb = _tc_dense([(p2, hs2)], dis, b2.reshape(1, 128),
                                       W3, [128, 128])

  p3a = _make_sc_scatter(128)(_interleave(hs3ab), srcp, dstp, zer128)
  p3b = _make_sc_scatter(128)(_interleave(hs3bb), srcp, dstp, zer128)
  hs4, hs4b = _tc_dense([(p3a, hs3a), (p3b, hs3b)], dis, b3.reshape(1, 256),
                        W4, [128])

  p4 = _make_sc_scatter(128)(_interleave(hs4b), srcp, dstp, zer128)
  out = _tc_final(p4, hs4, dis, b4.reshape(1, 128),
                  fw1, fb1.reshape(1, 256), fw2, fb2.reshape(1, 128),
                  fw3, fb3.reshape(1, 128))
  return out[:N]


# speedup vs baseline: 1.7101x; 1.0789x over previous
import functools

import jax
import jax.numpy as jnp
from jax import lax
from jax.experimental import pallas as pl
from jax.experimental.pallas import tpu as pltpu
from jax.experimental.pallas import tpu_sc as plsc

N = 10000
E = 320000
D = 128

NC = 2
NS = 16
NW = NC * NS
CHUNK = 128
EPAD = 327680
NCHUNKS = EPAD // CHUNK
CPW0 = 100
CPW1 = 60
SLABR = CPW0 + 1
NPAD = 10240
RPS = NPAD // NS

BR = 640
GRID = NPAD // BR

_f32 = jnp.float32
_bf16 = jnp.bfloat16


def _widen_rows(bf_ref, f_ref, F):
  mask = jnp.int32(-65536)

  @plsc.parallel_loop(0, CHUNK, 1, unroll=4)
  def _row(r):
    for u in range(F // 32):
      v = plsc.bitcast(bf_ref[r, pl.ds(u * 32, 32)], jnp.int32)
      lo = plsc.bitcast(lax.shift_left(v, 16), _f32)
      hi = plsc.bitcast(jnp.bitwise_and(v, mask), _f32)
      f_ref[r, pl.ds(u * 32, 16)] = lo
      f_ref[r, pl.ds(u * 32 + 16, 16)] = hi


@functools.lru_cache(maxsize=None)
def _make_sc_scatter(F):
  mesh = plsc.VectorSubcoreMesh(core_axis_name="c", subcore_axis_name="s",
                                num_cores=NC, num_subcores=NS)

  @functools.partial(
      pl.kernel,
      out_type=jax.ShapeDtypeStruct((NC, NPAD, F), _f32),
      mesh=mesh,
      scratch_types=[
          pltpu.VMEM((CPW0 // 2 + 1, CHUNK), jnp.int32),
          pltpu.VMEM((CPW0 // 2, CHUNK), jnp.int32),
          pltpu.VMEM((CHUNK, F), _bf16),
          pltpu.VMEM((CHUNK, F), _bf16),
          pltpu.VMEM((CHUNK, F), _f32),
          pltpu.VMEM_SHARED((NPAD, F), _f32),
          pltpu.SemaphoreType.DMA,
          pltpu.SemaphoreType.DMA,
      ],
      compiler_params=pltpu.CompilerParams(use_tc_tiling_on_sc=False,
                                           needs_layout_passes=False),
  )
  def k(h_hbm, srcp_hbm, dstp_hbm, zer_hbm, out_hbm, src_v, dst_v, bf0, bf1,
        rows_f, acc, sem0, sem1):
    c = lax.axis_index("c")
    s = lax.axis_index("s")
    wid = s * NC + c
    half = jnp.where(c == 0, CPW0 // 2, CPW1 // 2)
    pltpu.sync_copy(zer_hbm, acc.at[pl.ds(s * RPS, RPS)])
    plsc.subcore_barrier()

    for ph in range(2):
      base = ph * half
      pltpu.sync_copy(srcp_hbm.at[wid, pl.ds(base, CPW0 // 2 + 1)], src_v)
      pltpu.sync_copy(dstp_hbm.at[wid, pl.ds(base, CPW0 // 2)], dst_v)

      pltpu.async_copy(h_hbm.at[src_v.at[0]], bf0, sem0)

      def body(jj, carry):
        j0 = jj * 2
        pltpu.async_copy(h_hbm.at[src_v.at[j0 + 1]], bf1, sem1)
        pltpu.make_async_copy(h_hbm.at[src_v.at[j0]], bf0, sem0).wait()
        _widen_rows(bf0, rows_f, F)
        pltpu.sync_copy(rows_f, acc.at[dst_v.at[j0]], add=True)
        pltpu.async_copy(h_hbm.at[src_v.at[j0 + 2]], bf0, sem0)
        pltpu.make_async_copy(h_hbm.at[src_v.at[j0 + 1]], bf1, sem1).wait()
        _widen_rows(bf1, rows_f, F)
        pltpu.sync_copy(rows_f, acc.at[dst_v.at[j0 + 1]], add=True)
        return carry

      lax.fori_loop(0, half // 2, body, 0)
      pltpu.make_async_copy(h_hbm.at[src_v.at[half]], bf0, sem0).wait()

    plsc.subcore_barrier()
    pltpu.sync_copy(acc.at[pl.ds(s * RPS, RPS)],
                    out_hbm.at[c, pl.ds(s * RPS, RPS)])

  return k


_DEGW = 16


@functools.lru_cache(maxsize=None)
def _make_sc_deg():
  mesh = plsc.VectorSubcoreMesh(core_axis_name="c", subcore_axis_name="s",
                                num_cores=NC, num_subcores=NS)

  @functools.partial(
      pl.kernel,
      out_type=jax.ShapeDtypeStruct((NC, NPAD, _DEGW), _f32),
      mesh=mesh,
      scratch_types=[
          pltpu.VMEM((CPW0, CHUNK), jnp.int32),
          pltpu.VMEM((CHUNK, _DEGW), _f32),
          pltpu.VMEM_SHARED((NPAD, _DEGW), _f32),
      ],
      compiler_params=pltpu.CompilerParams(use_tc_tiling_on_sc=False),
  )
  def k(dstp_hbm, ones_hbm, zer_hbm, out_hbm, dst_v, ones_v, acc):
    c = lax.axis_index("c")
    s = lax.axis_index("s")
    wid = s * NC + c
    pltpu.sync_copy(dstp_hbm.at[wid, pl.ds(0, CPW0)], dst_v)
    pltpu.sync_copy(ones_hbm, ones_v)
    pltpu.sync_copy(zer_hbm, acc.at[pl.ds(s * RPS, RPS)])
    plsc.subcore_barrier()

    cnt = jnp.where(c == 0, CPW0, CPW1)

    def body(j, carry):
      pltpu.sync_copy(ones_v, acc.at[dst_v.at[j]], add=True)
      return carry

    lax.fori_loop(0, cnt, body, 0)
    plsc.subcore_barrier()
    pltpu.sync_copy(acc.at[pl.ds(s * RPS, RPS)],
                    out_hbm.at[c, pl.ds(s * RPS, RPS)])

  return k


def _full(shape):
  return pl.BlockSpec(shape, lambda i: tuple(0 for _ in shape))


def _rows(shape):
  if len(shape) == 3:
    return pl.BlockSpec(shape, lambda i: (0, i, 0))
  return pl.BlockSpec(shape, lambda i: (i, 0))


def _interleave(hs_bf):
  n, f = hs_bf.shape
  return hs_bf.reshape(n, f // 32, 2, 16).swapaxes(-1, -2).reshape(n, f)


def _tc_pre(x, t, deg_p, w1a, w1b):
  def body(x_ref, t_ref, dp_ref, wa_ref, wb_ref, dis_ref, hs_ref, hsb_ref):
    dp = dp_ref[...]
    deg = dp[0][:, 0:1] + dp[1][:, 0:1] + 1.0
    dis = lax.rsqrt(deg)
    h = jnp.dot(x_ref[...], wa_ref[...], preferred_element_type=_f32)
    h = h + t_ref[...] * wb_ref[...]
    hs = dis * h
    dis_ref[...] = dis
    hs_ref[...] = hs
    hsb_ref[...] = hs.astype(_bf16)

  return pl.pallas_call(
      body,
      grid=(GRID,),
      in_specs=[
          _rows((BR, D)),
          _rows((BR, 1)),
          _rows((NC, BR, _DEGW)),
          _full((D, 64)),
          _full((1, 64)),
      ],
      out_specs=[_rows((BR, 1)), _rows((BR, 64)), _rows((BR, 64))],
      out_shape=[
          jax.ShapeDtypeStruct((NPAD, 1), _f32),
          jax.ShapeDtypeStruct((NPAD, 64), _f32),
          jax.ShapeDtypeStruct((NPAD, 64), _bf16),
      ],
  )(x, t, deg_p, w1a, w1b)


def _tc_dense(parts, dis, b, w, fouts):
  fins = [hs.shape[1] for _, hs in parts]
  fin = sum(fins)
  fout = sum(fouts)
  n_parts = len(parts)
  n_out = len(fouts)

  def body(*refs):
    in_refs = refs[:2 * n_parts]
    dis_ref, b_ref, w_ref = refs[2 * n_parts:2 * n_parts + 3]
    out_refs = refs[2 * n_parts + 3:]
    dis = dis_ref[...]
    segs = []
    for i in range(n_parts):
      p = in_refs[2 * i][...]
      hs = in_refs[2 * i + 1][...]
      segs.append(p[0] + p[1] + hs)
    agg = segs[0] if n_parts == 1 else jnp.concatenate(segs, axis=1)
    a = jax.nn.gelu(dis * agg + b_ref[...])
    h = jnp.dot(a, w_ref[...], preferred_element_type=_f32)
    hs_out = dis * h
    off = 0
    for i, f in enumerate(fouts):
      blk = hs_out[:, off:off + f]
      out_refs[i][...] = blk
      out_refs[n_out + i][...] = blk.astype(_bf16)
      off += f

  in_specs = []
  args = []
  for p, hs in parts:
    f = hs.shape[1]
    in_specs += [_rows((NC, BR, f)), _rows((BR, f))]
    args += [p, hs]
  in_specs += [_rows((BR, 1)), _full((1, fin)), _full((fin, fout))]
  args += [dis, b, w]

  return pl.pallas_call(
      body,
      grid=(GRID,),
      in_specs=in_specs,
      out_specs=([_rows((BR, f)) for f in fouts] +
                 [_rows((BR, f)) for f in fouts]),
      out_shape=([jax.ShapeDtypeStruct((NPAD, f), _f32) for f in fouts] +
                 [jax.ShapeDtypeStruct((NPAD, f), _bf16) for f in fouts]),
  )(*args)


def _tc_final(p4, hs4, dis, b4, fw1, fb1, fw2, fb2, fw3, fb3):
  def body(p_ref, hs_ref, dis_ref, b_ref, w1_ref, c1_ref, w2_ref, c2_ref,
           w3_ref, c3_ref, out_ref):
    dis = dis_ref[...]
    p = p_ref[...]
    a = jax.nn.gelu(dis * (p[0] + p[1] + hs_ref[...]) + b_ref[...])
    z = jax.nn.gelu(
        jnp.dot(a, w1_ref[...], preferred_element_type=_f32) + c1_ref[...])
    z = jax.nn.gelu(
        jnp.dot(z, w2_ref[...], preferred_element_type=_f32) + c2_ref[...])
    out_ref[...] = (
        jnp.dot(z, w3_ref[...], preferred_element_type=_f32) + c3_ref[...])

  return pl.pallas_call(
      body,
      grid=(GRID,),
      in_specs=[
          _rows((NC, BR, 128)),
          _rows((BR, 128)),
          _rows((BR, 1)),
          _full((1, 128)),
          _full((128, 256)),
          _full((1, 256)),
          _full((256, 128)),
          _full((1, 128)),
          _full((128, 128)),
          _full((1, 128)),
      ],
      out_specs=_rows((BR, 128)),
      out_shape=jax.ShapeDtypeStruct((NPAD, 128), _f32),
  )(p4, hs4, dis, b4, fw1, fb1, fw2, fb2, fw3, fb3)


def kernel(x, adj, t, W1, b1, W2, b2, W3, b3, W4, b4,
           fw1, fb1, fw2, fb2, fw3, fb3):
  pad_i = jnp.full((EPAD - E,), N, dtype=jnp.int32)

  def _slab(idx):
    ch = jnp.concatenate([idx, pad_i]).reshape(NCHUNKS, CHUNK)
    slabs = []
    off = 0
    for w in range(NW):
      cnt = CPW0 if w % NC == 0 else CPW1
      blk = ch[off:off + cnt]
      off += cnt
      fill = jnp.full((SLABR - cnt, CHUNK), N, dtype=jnp.int32)
      slabs.append(jnp.concatenate([blk, fill]))
    return jnp.stack(slabs)

  srcp = _slab(adj[0])
  dstp = _slab(adj[1])

  ones_w = jnp.ones((CHUNK, _DEGW), _f32)
  zer_w = jnp.zeros((RPS, _DEGW), _f32)
  zer64 = jnp.zeros((RPS, 64), _f32)
  zer128 = jnp.zeros((RPS, 128), _f32)

  xp = jnp.zeros((NPAD, D), _f32).at[:N].set(x.astype(_f32))
  tp = jnp.zeros((NPAD, 1), _f32).at[:N, 0].set(t.astype(_f32))

  deg_p = _make_sc_deg()(dstp, ones_w, zer_w)
  dis, hs1, hs1b = _tc_pre(xp, tp, deg_p, W1[:D], W1[D:].reshape(1, 64))

  p1 = _make_sc_scatter(64)(_interleave(hs1b), srcp, dstp, zer64)
  hs2, hs2b = _tc_dense([(p1, hs1)], dis, b1.reshape(1, 64), W2, [128])

  p2 = _make_sc_scatter(128)(_interleave(hs2b), srcp, dstp, zer128)
  hs3a, hs3b, hs3ab, hs3bb = _tc_dense([(p2, hs2)], dis, b2.reshape(1, 128),
                                       W3, [128, 128])

  p3a = _make_sc_scatter(128)(_interleave(hs3ab), srcp, dstp, zer128)
  p3b = _make_sc_scatter(128)(_interleave(hs3bb), srcp, dstp, zer128)
  hs4, hs4b = _tc_dense([(p3a, hs3a), (p3b, hs3b)], dis, b3.reshape(1, 256),
                        W4, [128])

  p4 = _make_sc_scatter(128)(_interleave(hs4b), srcp, dstp, zer128)
  out = _tc_final(p4, hs4, dis, b4.reshape(1, 128),
                  fw1, fb1.reshape(1, 256), fw2, fb2.reshape(1, 128),
                  fw3, fb3.reshape(1, 128))
  return out[:N]


# speedup vs baseline: 1.7419x; 1.0186x over previous
import functools

import jax
import jax.numpy as jnp
from jax import lax
from jax.experimental import pallas as pl
from jax.experimental.pallas import tpu as pltpu
from jax.experimental.pallas import tpu_sc as plsc

N = 10000
E = 320000
D = 128

NC = 2
NS = 16
NW = NC * NS
CHUNK = 128
EPAD = 327680
NCHUNKS = EPAD // CHUNK
CPW0 = 100
CPW1 = 60
SLABR = CPW0 + 1
NPAD = 10240
RPS = NPAD // NS

BR = 640
GRID = NPAD // BR

_f32 = jnp.float32
_bf16 = jnp.bfloat16


def _widen_rows(bf_ref, f_ref, F):
  mask = jnp.int32(-65536)

  @plsc.parallel_loop(0, CHUNK, 1, unroll=4)
  def _row(r):
    for u in range(F // 32):
      v = plsc.bitcast(bf_ref[r, pl.ds(u * 32, 32)], jnp.int32)
      lo = plsc.bitcast(lax.shift_left(v, 16), _f32)
      hi = plsc.bitcast(jnp.bitwise_and(v, mask), _f32)
      f_ref[r, pl.ds(u * 32, 16)] = lo
      f_ref[r, pl.ds(u * 32 + 16, 16)] = hi


@functools.lru_cache(maxsize=None)
def _make_sc_scatter(F):
  mesh = plsc.VectorSubcoreMesh(core_axis_name="c", subcore_axis_name="s",
                                num_cores=NC, num_subcores=NS)

  @functools.partial(
      pl.kernel,
      out_type=jax.ShapeDtypeStruct((NC, NPAD, F), _f32),
      mesh=mesh,
      scratch_types=[
          pltpu.VMEM((CPW0 // 2 + 1, CHUNK), jnp.int32),
          pltpu.VMEM((CPW0 // 2, CHUNK), jnp.int32),
          pltpu.VMEM((CHUNK, F), _bf16),
          pltpu.VMEM((CHUNK, F), _bf16),
          pltpu.VMEM((CHUNK, F), _f32),
          pltpu.VMEM_SHARED((NPAD, F), _f32),
          pltpu.SemaphoreType.DMA,
          pltpu.SemaphoreType.DMA,
      ],
      compiler_params=pltpu.CompilerParams(use_tc_tiling_on_sc=False,
                                           needs_layout_passes=False),
  )
  def k(h_hbm, srcp_hbm, dstp_hbm, out_hbm, src_v, dst_v, bf0, bf1,
        rows_f, acc, sem0, sem1):
    c = lax.axis_index("c")
    s = lax.axis_index("s")
    wid = s * NC + c
    half = jnp.where(c == 0, CPW0 // 2, CPW1 // 2)

    @plsc.parallel_loop(0, CHUNK, 1, unroll=4)
    def _zero_row(r):
      for u in range(F // 16):
        rows_f[r, pl.ds(u * 16, 16)] = jnp.zeros((16,), _f32)

    for q in range(RPS // CHUNK):
      pltpu.sync_copy(rows_f, acc.at[pl.ds(s * RPS + q * CHUNK, CHUNK)])
    plsc.subcore_barrier()

    for ph in range(2):
      base = ph * half
      pltpu.sync_copy(srcp_hbm.at[wid, pl.ds(base, CPW0 // 2 + 1)], src_v)
      pltpu.sync_copy(dstp_hbm.at[wid, pl.ds(base, CPW0 // 2)], dst_v)

      pltpu.async_copy(h_hbm.at[src_v.at[0]], bf0, sem0)

      def body(jj, carry):
        j0 = jj * 2
        pltpu.async_copy(h_hbm.at[src_v.at[j0 + 1]], bf1, sem1)
        pltpu.make_async_copy(h_hbm.at[src_v.at[j0]], bf0, sem0).wait()
        _widen_rows(bf0, rows_f, F)
        pltpu.sync_copy(rows_f, acc.at[dst_v.at[j0]], add=True)
        pltpu.async_copy(h_hbm.at[src_v.at[j0 + 2]], bf0, sem0)
        pltpu.make_async_copy(h_hbm.at[src_v.at[j0 + 1]], bf1, sem1).wait()
        _widen_rows(bf1, rows_f, F)
        pltpu.sync_copy(rows_f, acc.at[dst_v.at[j0 + 1]], add=True)
        return carry

      lax.fori_loop(0, half // 2, body, 0)
      pltpu.make_async_copy(h_hbm.at[src_v.at[half]], bf0, sem0).wait()

    plsc.subcore_barrier()
    pltpu.sync_copy(acc.at[pl.ds(s * RPS, RPS)],
                    out_hbm.at[c, pl.ds(s * RPS, RPS)])

  return k


_DEGW = 16


@functools.lru_cache(maxsize=None)
def _make_sc_deg():
  mesh = plsc.VectorSubcoreMesh(core_axis_name="c", subcore_axis_name="s",
                                num_cores=NC, num_subcores=NS)

  @functools.partial(
      pl.kernel,
      out_type=jax.ShapeDtypeStruct((NC, NPAD, _DEGW), _f32),
      mesh=mesh,
      scratch_types=[
          pltpu.VMEM((CPW0, CHUNK), jnp.int32),
          pltpu.VMEM((CHUNK, _DEGW), _f32),
          pltpu.VMEM((CHUNK, _DEGW), _f32),
          pltpu.VMEM_SHARED((NPAD, _DEGW), _f32),
      ],
      compiler_params=pltpu.CompilerParams(use_tc_tiling_on_sc=False,
                                           needs_layout_passes=False),
  )
  def k(dstp_hbm, out_hbm, dst_v, ones_v, zro_v, acc):
    c = lax.axis_index("c")
    s = lax.axis_index("s")
    wid = s * NC + c
    pltpu.sync_copy(dstp_hbm.at[wid, pl.ds(0, CPW0)], dst_v)

    @plsc.parallel_loop(0, CHUNK, 1, unroll=4)
    def _fill_row(r):
      ones_v[r, pl.ds(0, 16)] = jnp.ones((16,), _f32)
      zro_v[r, pl.ds(0, 16)] = jnp.zeros((16,), _f32)

    for q in range(RPS // CHUNK):
      pltpu.sync_copy(zro_v, acc.at[pl.ds(s * RPS + q * CHUNK, CHUNK)])
    plsc.subcore_barrier()

    cnt = jnp.where(c == 0, CPW0, CPW1)

    def body(j, carry):
      pltpu.sync_copy(ones_v, acc.at[dst_v.at[j]], add=True)
      return carry

    lax.fori_loop(0, cnt, body, 0)
    plsc.subcore_barrier()
    pltpu.sync_copy(acc.at[pl.ds(s * RPS, RPS)],
                    out_hbm.at[c, pl.ds(s * RPS, RPS)])

  return k


def _full(shape):
  return pl.BlockSpec(shape, lambda i: tuple(0 for _ in shape))


def _rows(shape):
  if len(shape) == 3:
    return pl.BlockSpec(shape, lambda i: (0, i, 0))
  return pl.BlockSpec(shape, lambda i: (i, 0))


def _interleave(hs_bf):
  n, f = hs_bf.shape
  return hs_bf.reshape(n, f // 32, 2, 16).swapaxes(-1, -2).reshape(n, f)


def _tc_pre(x, t, deg_p, w1a, w1b):
  def body(x_ref, t_ref, dp_ref, wa_ref, wb_ref, dis_ref, hs_ref, hsb_ref):
    dp = dp_ref[...]
    deg = dp[0][:, 0:1] + dp[1][:, 0:1] + 1.0
    dis = lax.rsqrt(deg)
    h = jnp.dot(x_ref[...], wa_ref[...], preferred_element_type=_f32)
    h = h + t_ref[...] * wb_ref[...]
    hs = dis * h
    dis_ref[...] = dis
    hs_ref[...] = hs
    hsb_ref[...] = hs.astype(_bf16)

  return pl.pallas_call(
      body,
      grid=(GRID,),
      in_specs=[
          _rows((BR, D)),
          _rows((BR, 1)),
          _rows((NC, BR, _DEGW)),
          _full((D, 64)),
          _full((1, 64)),
      ],
      out_specs=[_rows((BR, 1)), _rows((BR, 64)), _rows((BR, 64))],
      out_shape=[
          jax.ShapeDtypeStruct((NPAD, 1), _f32),
          jax.ShapeDtypeStruct((NPAD, 64), _f32),
          jax.ShapeDtypeStruct((NPAD, 64), _bf16),
      ],
  )(x, t, deg_p, w1a, w1b)


def _tc_dense(parts, dis, b, w, fouts):
  fins = [hs.shape[1] for _, hs in parts]
  fin = sum(fins)
  fout = sum(fouts)
  n_parts = len(parts)
  n_out = len(fouts)

  def body(*refs):
    in_refs = refs[:2 * n_parts]
    dis_ref, b_ref, w_ref = refs[2 * n_parts:2 * n_parts + 3]
    out_refs = refs[2 * n_parts + 3:]
    dis = dis_ref[...]
    segs = []
    for i in range(n_parts):
      p = in_refs[2 * i][...]
      hs = in_refs[2 * i + 1][...]
      segs.append(p[0] + p[1] + hs)
    agg = segs[0] if n_parts == 1 else jnp.concatenate(segs, axis=1)
    a = jax.nn.gelu(dis * agg + b_ref[...])
    h = jnp.dot(a, w_ref[...], preferred_element_type=_f32)
    hs_out = dis * h
    off = 0
    for i, f in enumerate(fouts):
      blk = hs_out[:, off:off + f]
      out_refs[i][...] = blk
      out_refs[n_out + i][...] = blk.astype(_bf16)
      off += f

  in_specs = []
  args = []
  for p, hs in parts:
    f = hs.shape[1]
    in_specs += [_rows((NC, BR, f)), _rows((BR, f))]
    args += [p, hs]
  in_specs += [_rows((BR, 1)), _full((1, fin)), _full((fin, fout))]
  args += [dis, b, w]

  return pl.pallas_call(
      body,
      grid=(GRID,),
      in_specs=in_specs,
      out_specs=([_rows((BR, f)) for f in fouts] +
                 [_rows((BR, f)) for f in fouts]),
      out_shape=([jax.ShapeDtypeStruct((NPAD, f), _f32) for f in fouts] +
                 [jax.ShapeDtypeStruct((NPAD, f), _bf16) for f in fouts]),
  )(*args)


def _tc_final(p4, hs4, dis, b4, fw1, fb1, fw2, fb2, fw3, fb3):
  def body(p_ref, hs_ref, dis_ref, b_ref, w1_ref, c1_ref, w2_ref, c2_ref,
           w3_ref, c3_ref, out_ref):
    dis = dis_ref[...]
    p = p_ref[...]
    a = jax.nn.gelu(dis * (p[0] + p[1] + hs_ref[...]) + b_ref[...])
    z = jax.nn.gelu(
        jnp.dot(a, w1_ref[...], preferred_element_type=_f32) + c1_ref[...])
    z = jax.nn.gelu(
        jnp.dot(z, w2_ref[...], preferred_element_type=_f32) + c2_ref[...])
    out_ref[...] = (
        jnp.dot(z, w3_ref[...], preferred_element_type=_f32) + c3_ref[...])

  return pl.pallas_call(
      body,
      grid=(GRID,),
      in_specs=[
          _rows((NC, BR, 128)),
          _rows((BR, 128)),
          _rows((BR, 1)),
          _full((1, 128)),
          _full((128, 256)),
          _full((1, 256)),
          _full((256, 128)),
          _full((1, 128)),
          _full((128, 128)),
          _full((1, 128)),
      ],
      out_specs=_rows((BR, 128)),
      out_shape=jax.ShapeDtypeStruct((NPAD, 128), _f32),
  )(p4, hs4, dis, b4, fw1, fb1, fw2, fb2, fw3, fb3)


def kernel(x, adj, t, W1, b1, W2, b2, W3, b3, W4, b4,
           fw1, fb1, fw2, fb2, fw3, fb3):
  pad_i = jnp.full((EPAD - E,), N, dtype=jnp.int32)

  def _slab(idx):
    ch = jnp.concatenate([idx, pad_i]).reshape(NCHUNKS, CHUNK)
    slabs = []
    off = 0
    for w in range(NW):
      cnt = CPW0 if w % NC == 0 else CPW1
      blk = ch[off:off + cnt]
      off += cnt
      fill = jnp.full((SLABR - cnt, CHUNK), N, dtype=jnp.int32)
      slabs.append(jnp.concatenate([blk, fill]))
    return jnp.stack(slabs)

  srcp = _slab(adj[0])
  dstp = _slab(adj[1])

  xp = jnp.zeros((NPAD, D), _f32).at[:N].set(x.astype(_f32))
  tp = jnp.zeros((NPAD, 1), _f32).at[:N, 0].set(t.astype(_f32))

  deg_p = _make_sc_deg()(dstp)
  dis, hs1, hs1b = _tc_pre(xp, tp, deg_p, W1[:D], W1[D:].reshape(1, 64))

  p1 = _make_sc_scatter(64)(_interleave(hs1b), srcp, dstp)
  hs2, hs2b = _tc_dense([(p1, hs1)], dis, b1.reshape(1, 64), W2, [128])

  p2 = _make_sc_scatter(128)(_interleave(hs2b), srcp, dstp)
  hs3a, hs3b, hs3ab, hs3bb = _tc_dense([(p2, hs2)], dis, b2.reshape(1, 128),
                                       W3, [128, 128])

  p3a = _make_sc_scatter(128)(_interleave(hs3ab), srcp, dstp)
  p3b = _make_sc_scatter(128)(_interleave(hs3bb), srcp, dstp)
  hs4, hs4b = _tc_dense([(p3a, hs3a), (p3b, hs3b)], dis, b3.reshape(1, 256),
                        W4, [128])

  p4 = _make_sc_scatter(128)(_interleave(hs4b), srcp, dstp)
  out = _tc_final(p4, hs4, dis, b4.reshape(1, 128),
                  fw1, fb1.reshape(1, 256), fw2, fb2.reshape(1, 128),
                  fw3, fb3.reshape(1, 128))
  return out[:N]
